# baseline (speedup 1.0000x reference)
import jax
import jax.numpy as jnp
from jax.experimental import pallas as pl
from jax.experimental.pallas import tpu as pltpu

N = 512
D = 64
R = 32
RI = 2
BI = 16
CX = 4
RX = 64
W_PACK = 128
NEG = -1e30


def _gcn_block(cc_ref, h_ref, c_ref, og_ref, relw_ref, relb_ref,
               wcomb_ref, whi_ref, whj_ref, bias_ref, wnei_ref, wneib_ref,
               hout_ref, cout_ref):
    i = pl.program_id(0)
    bf16 = jnp.bfloat16
    f32 = jnp.float32

    cc = cc_ref[...].reshape(BI * N, CX).astype(bf16)
    rx = jnp.dot(cc, relw_ref[...], preferred_element_type=f32)
    rx = jnp.maximum(rx + relb_ref[...], 0.0)

    logits = jnp.dot(rx.astype(bf16), wcomb_ref[...],
                     preferred_element_type=f32)

    h_all = h_ref[...]
    h_blk = h_ref[pl.ds(i * BI, BI), :]
    a_i = jnp.dot(h_blk, whi_ref[...], preferred_element_type=f32)
    a_i = a_i + bias_ref[...]
    b_j = jnp.dot(h_all, whj_ref[...], preferred_element_type=f32)

    lg = logits.reshape(BI, N, W_PACK) + a_i[:, None, :] + b_j[None, :, :]

    gate = jax.nn.sigmoid(lg[:, :, :D])
    tt = lg[:, :, D:D + 1]
    mflag = lg[:, :, D + 1:D + 2]

    m2 = (mflag > 0.5) & (tt != 0.0)
    mx = jnp.max(jnp.where(m2, tt, NEG), axis=1, keepdims=True)
    w = jnp.where(m2, jnp.exp(tt - mx), 0.0)
    s = jnp.sum(w, axis=1, keepdims=True)
    p = w / jnp.where(s > 0.0, s, 1.0)

    h_sum = jnp.sum(gate * (p * h_all[None, :, :]), axis=1)

    c_out = jnp.dot(h_sum, wnei_ref[...], preferred_element_type=f32)
    c_out = c_out + wneib_ref[...] + c_ref[...]
    cout_ref[...] = c_out
    hout_ref[...] = og_ref[...] * jnp.tanh(c_out)


def kernel(corr_index, nei_index, nei_num, outgate, self_h, self_c,
           rel_W, rel_b, ngate_W, ngate_b, war_W, war_b, wnei_W, wnei_b):
    n = corr_index.shape[0]
    d = self_h.shape[1]
    ri = corr_index.shape[2]
    r = rel_W.shape[1]
    assert (n, d, ri, r) == (N, D, RI, R)
    f32 = jnp.float32

    mask_col = (nei_index > 0).astype(f32)[:, :, None]
    pad_col = jnp.zeros((n, n, CX - ri - 1), f32)
    ccx = jnp.concatenate([corr_index, mask_col, pad_col], axis=2)

    relw = jnp.zeros((CX, RX), f32).at[:ri, :r].set(rel_W).at[ri, r].set(1.0)
    relb = jnp.zeros((1, RX), f32).at[0, :r].set(rel_b)

    wcomb = (jnp.zeros((RX, W_PACK), f32)
             .at[:r, :d].set(ngate_W[:r])
             .at[:r, d:d + 1].set(war_W[:r])
             .at[r, d + 1].set(1.0))
    whi = jnp.zeros((d, W_PACK), f32).at[:, :d].set(ngate_W[r:r + d]) \
        .at[:, d:d + 1].set(war_W[r:r + d])
    whj = jnp.zeros((d, W_PACK), f32).at[:, :d].set(ngate_W[r + d:]) \
        .at[:, d:d + 1].set(war_W[r + d:])
    bias = jnp.zeros((1, W_PACK), f32).at[0, :d].set(ngate_b) \
        .at[0, d].set(war_b[0])

    grid = (n // BI,)
    full = lambda shape: pl.BlockSpec(shape, lambda i: (0,) * len(shape))
    row_blk = lambda shape: pl.BlockSpec(shape, lambda i: (i,) + (0,) * (len(shape) - 1))

    h_out, c_out = pl.pallas_call(
        _gcn_block,
        grid=grid,
        in_specs=[
            row_blk((BI, n, CX)),
            full((n, d)),
            row_blk((BI, d)),
            row_blk((BI, d)),
            full((CX, RX)),
            full((1, RX)),
            full((RX, W_PACK)),
            full((d, W_PACK)),
            full((d, W_PACK)),
            full((1, W_PACK)),
            full((d, d)),
            full((1, d)),
        ],
        out_specs=[row_blk((BI, d)), row_blk((BI, d))],
        out_shape=[
            jax.ShapeDtypeStruct((n, d), f32),
            jax.ShapeDtypeStruct((n, d), f32),
        ],
        compiler_params=pltpu.CompilerParams(
            dimension_semantics=("arbitrary",),
        ),
    )(ccx, self_h, self_c, outgate,
      relw.astype(jnp.bfloat16), relb, wcomb.astype(jnp.bfloat16), whi, whj,
      bias, wnei_W, wnei_b.reshape(1, d))

    return (outgate, h_out, c_out)


# speedup vs baseline: 2.7693x; 2.7693x over previous
import jax
import jax.numpy as jnp
from jax.experimental import pallas as pl
from jax.experimental.pallas import tpu as pltpu

N = 512
D = 64
R = 32
RI = 2
BI = 16
NEG = -1e30


def _gcn_block(cc_ref, c0_ref, c1_ref, nei_ref, h_ref, ht_ref, c_ref, og_ref,
               relw_ref, relb_ref, wgr_ref, whi_ref, whj_ref, gbias_ref,
               rels_ref, wars_ref, warhi_ref, warhjt_ref, wnei_ref, wneib_ref,
               hout_ref, cout_ref):
    i = pl.program_id(0)
    bf16 = jnp.bfloat16
    f32 = jnp.float32

    cc = cc_ref[...].reshape(BI * N, RI).astype(bf16)
    r = jnp.dot(cc, relw_ref[...], preferred_element_type=f32)
    r = jnp.maximum(r + relb_ref[...], 0.0).astype(bf16)
    glog = jnp.dot(r, wgr_ref[...], preferred_element_type=f32)

    h_all = h_ref[...]
    h_blk = h_ref[pl.ds(i * BI, BI), :]
    a_i = jnp.dot(h_blk, whi_ref[...], preferred_element_type=f32)
    a_i = a_i + gbias_ref[...]
    b_j = jnp.dot(h_all, whj_ref[...], preferred_element_type=f32)

    lg = (glog.reshape(BI, N, D) + a_i[:, None, :] + b_j[None, :, :])
    gate = jax.nn.sigmoid(lg.astype(bf16))
    q = gate * h_all[None, :, :]

    c0 = c0_ref[...]
    c1 = c1_ref[...]
    t = jnp.zeros((BI, N), f32)
    for k in range(R):
        rk = jnp.maximum(c0 * rels_ref[0, k] + c1 * rels_ref[1, k]
                         + rels_ref[2, k], 0.0)
        t = t + rk * wars_ref[0, k]
    aw = jnp.dot(h_blk.astype(f32), warhi_ref[...],
                 preferred_element_type=f32)
    bw = jnp.dot(warhjt_ref[...], ht_ref[...],
                 preferred_element_type=f32)
    tt = t + aw + bw + wars_ref[1, 0]

    m2 = (nei_ref[...] > 0) & (tt != 0.0)
    mx = jnp.max(jnp.where(m2, tt, NEG), axis=1, keepdims=True)
    w = jnp.where(m2, jnp.exp(tt - mx), 0.0)
    s = jnp.sum(w, axis=1, keepdims=True)
    p = (w / jnp.where(s > 0.0, s, 1.0)).astype(bf16)

    h_sum = jax.lax.dot_general(
        p, q, (((1,), (1,)), ((0,), (0,))),
        preferred_element_type=f32)

    c_out = jnp.dot(h_sum, wnei_ref[...], preferred_element_type=f32)
    c_out = c_out + wneib_ref[...] + c_ref[...]
    cout_ref[...] = c_out
    hout_ref[...] = og_ref[...] * jnp.tanh(c_out)


def kernel(corr_index, nei_index, nei_num, outgate, self_h, self_c,
           rel_W, rel_b, ngate_W, ngate_b, war_W, war_b, wnei_W, wnei_b):
    n = corr_index.shape[0]
    d = self_h.shape[1]
    ri = corr_index.shape[2]
    r = rel_W.shape[1]
    assert (n, d, ri, r) == (N, D, RI, R)
    f32 = jnp.float32
    bf16 = jnp.bfloat16

    c0 = corr_index[:, :, 0]
    c1 = corr_index[:, :, 1]

    rels = jnp.stack([rel_W[0], rel_W[1], rel_b])
    wars = jnp.zeros((2, r), f32).at[0].set(war_W[:r, 0]).at[1, 0].set(war_b[0])

    grid = (n // BI,)
    full = lambda shape: pl.BlockSpec(shape, lambda i: (0,) * len(shape))
    row_blk = lambda shape: pl.BlockSpec(shape, lambda i: (i,) + (0,) * (len(shape) - 1))
    smem = lambda shape: pl.BlockSpec(shape, lambda i: (0,) * len(shape),
                                      memory_space=pltpu.SMEM)

    h_out, c_out = pl.pallas_call(
        _gcn_block,
        grid=grid,
        in_specs=[
            row_blk((BI, n, ri)),
            row_blk((BI, n)),
            row_blk((BI, n)),
            row_blk((BI, n)),
            full((n, d)),
            full((d, n)),
            row_blk((BI, d)),
            row_blk((BI, d)),
            full((ri, r)),
            full((1, r)),
            full((r, d)),
            full((d, d)),
            full((d, d)),
            full((1, d)),
            smem((3, r)),
            smem((2, r)),
            full((d, 1)),
            full((1, d)),
            full((d, d)),
            full((1, d)),
        ],
        out_specs=[row_blk((BI, d)), row_blk((BI, d))],
        out_shape=[
            jax.ShapeDtypeStruct((n, d), f32),
            jax.ShapeDtypeStruct((n, d), f32),
        ],
        compiler_params=pltpu.CompilerParams(
            dimension_semantics=("arbitrary",),
        ),
    )(corr_index, c0, c1, nei_index,
      self_h.astype(bf16), self_h.T, self_c, outgate,
      rel_W.astype(bf16), rel_b.reshape(1, r),
      ngate_W[:r].astype(bf16), ngate_W[r:r + d].astype(bf16),
      ngate_W[r + d:].astype(bf16), ngate_b.reshape(1, d),
      rels, wars, war_W[r:r + d], war_W[r + d:].T,
      wnei_W, wnei_b.reshape(1, d))

    return (outgate, h_out, c_out)


# speedup vs baseline: 2.8423x; 1.0264x over previous
import jax
import jax.numpy as jnp
from jax.experimental import pallas as pl
from jax.experimental.pallas import tpu as pltpu

N = 512
D = 64
R = 32
RI = 2
BI = 32
NEG = -1e30


def _gcn_block(cc_ref, c0_ref, c1_ref, nei_ref, h_ref, ht_ref, c_ref, og_ref,
               relw_ref, relb_ref, wgr_ref, whi_ref, whj_ref, gbias_ref,
               rels_ref, wars_ref, warhi_ref, warhjt_ref, wnei_ref, wneib_ref,
               hout_ref, cout_ref):
    i = pl.program_id(0)
    bf16 = jnp.bfloat16
    f32 = jnp.float32

    cc = cc_ref[...].reshape(BI * N, RI).astype(bf16)
    r = jnp.dot(cc, relw_ref[...], preferred_element_type=f32)
    r = jnp.maximum(r + relb_ref[...], 0.0).astype(bf16)
    glog = jnp.dot(r, wgr_ref[...], preferred_element_type=f32)

    h_all = h_ref[...]
    h_blk = h_ref[pl.ds(i * BI, BI), :]
    a_i = jnp.dot(h_blk, whi_ref[...], preferred_element_type=f32)
    a_i = a_i + gbias_ref[...]
    b_j = jnp.dot(h_all, whj_ref[...], preferred_element_type=f32)

    lg = (glog.reshape(BI, N, D) + a_i[:, None, :] + b_j[None, :, :])
    gate = jax.nn.sigmoid(lg.astype(bf16))
    q = gate * h_all[None, :, :]

    c0 = c0_ref[...]
    c1 = c1_ref[...]
    t = jnp.zeros((BI, N), f32)
    for k in range(R):
        rk = jnp.maximum(c0 * rels_ref[0, k] + c1 * rels_ref[1, k]
                         + rels_ref[2, k], 0.0)
        t = t + rk * wars_ref[0, k]
    aw = jnp.dot(h_blk.astype(f32), warhi_ref[...],
                 preferred_element_type=f32)
    bw = jnp.dot(warhjt_ref[...], ht_ref[...],
                 preferred_element_type=f32)
    tt = t + aw + bw + wars_ref[1, 0]

    m2 = (nei_ref[...] > 0) & (tt != 0.0)
    mx = jnp.max(jnp.where(m2, tt, NEG), axis=1, keepdims=True)
    w = jnp.where(m2, jnp.exp(tt - mx), 0.0)
    s = jnp.sum(w, axis=1, keepdims=True)
    p = (w / jnp.where(s > 0.0, s, 1.0)).astype(bf16)

    h_sum = jax.lax.dot_general(
        p, q, (((1,), (1,)), ((0,), (0,))),
        preferred_element_type=f32)

    c_out = jnp.dot(h_sum, wnei_ref[...], preferred_element_type=f32)
    c_out = c_out + wneib_ref[...] + c_ref[...]
    cout_ref[...] = c_out
    hout_ref[...] = og_ref[...] * jnp.tanh(c_out)


def kernel(corr_index, nei_index, nei_num, outgate, self_h, self_c,
           rel_W, rel_b, ngate_W, ngate_b, war_W, war_b, wnei_W, wnei_b):
    n = corr_index.shape[0]
    d = self_h.shape[1]
    ri = corr_index.shape[2]
    r = rel_W.shape[1]
    assert (n, d, ri, r) == (N, D, RI, R)
    f32 = jnp.float32
    bf16 = jnp.bfloat16

    c0 = corr_index[:, :, 0]
    c1 = corr_index[:, :, 1]

    rels = jnp.stack([rel_W[0], rel_W[1], rel_b])
    wars = jnp.zeros((2, r), f32).at[0].set(war_W[:r, 0]).at[1, 0].set(war_b[0])

    grid = (n // BI,)
    full = lambda shape: pl.BlockSpec(shape, lambda i: (0,) * len(shape))
    row_blk = lambda shape: pl.BlockSpec(shape, lambda i: (i,) + (0,) * (len(shape) - 1))
    smem = lambda shape: pl.BlockSpec(shape, lambda i: (0,) * len(shape),
                                      memory_space=pltpu.SMEM)

    h_out, c_out = pl.pallas_call(
        _gcn_block,
        grid=grid,
        in_specs=[
            row_blk((BI, n, ri)),
            row_blk((BI, n)),
            row_blk((BI, n)),
            row_blk((BI, n)),
            full((n, d)),
            full((d, n)),
            row_blk((BI, d)),
            row_blk((BI, d)),
            full((ri, r)),
            full((1, r)),
            full((r, d)),
            full((d, d)),
            full((d, d)),
            full((1, d)),
            smem((3, r)),
            smem((2, r)),
            full((d, 1)),
            full((1, d)),
            full((d, d)),
            full((1, d)),
        ],
        out_specs=[row_blk((BI, d)), row_blk((BI, d))],
        out_shape=[
            jax.ShapeDtypeStruct((n, d), f32),
            jax.ShapeDtypeStruct((n, d), f32),
        ],
        compiler_params=pltpu.CompilerParams(
            dimension_semantics=("arbitrary",),
        ),
    )(corr_index, c0, c1, nei_index,
      self_h.astype(bf16), self_h.T, self_c, outgate,
      rel_W.astype(bf16), rel_b.reshape(1, r),
      ngate_W[:r].astype(bf16), ngate_W[r:r + d].astype(bf16),
      ngate_W[r + d:].astype(bf16), ngate_b.reshape(1, d),
      rels, wars, war_W[r:r + d], war_W[r + d:].T,
      wnei_W, wnei_b.reshape(1, d))

    return (outgate, h_out, c_out)


# speedup vs baseline: 2.9002x; 1.0203x over previous
import jax
import jax.numpy as jnp
from jax.experimental import pallas as pl
from jax.experimental.pallas import tpu as pltpu

N = 512
D = 64
R = 32
RI = 2
BI = 32
NEG = -1e30


def _gcn_block(cc_ref, ct_ref, nei_ref, h_ref, ht_ref, c_ref, og_ref,
               relw_ref, relb_ref, ngw_ref, ngb_ref, war_ref, warhjt_ref,
               rels_ref, wars_ref, wnei_ref, wneib_ref,
               hout_ref, cout_ref):
    i = pl.program_id(0)
    bf16 = jnp.bfloat16
    f32 = jnp.float32

    cc = cc_ref[...].reshape(BI * N, RI).astype(bf16)
    r = jnp.dot(cc, relw_ref[...].astype(bf16), preferred_element_type=f32)
    r = jnp.maximum(r + relb_ref[...], 0.0).astype(bf16)
    ngw = ngw_ref[...].astype(bf16)
    glog = jnp.dot(r, ngw[:R], preferred_element_type=f32)

    h_all = h_ref[...].astype(bf16)
    h_blk = h_ref[pl.ds(i * BI, BI), :]
    a_i = jnp.dot(h_blk.astype(bf16), ngw[R:R + D],
                  preferred_element_type=f32)
    a_i = a_i + ngb_ref[...]
    b_j = jnp.dot(h_all, ngw[R + D:], preferred_element_type=f32)

    lg = (glog.reshape(BI, N, D) + a_i[:, None, :] + b_j[None, :, :])
    gate = jax.nn.sigmoid(lg.astype(bf16))
    q = gate * h_all[None, :, :]

    c0 = ct_ref[0]
    c1 = ct_ref[1]
    t = jnp.zeros((BI, N), f32)
    for k in range(R):
        rk = jnp.maximum(c0 * rels_ref[0, k] + c1 * rels_ref[1, k]
                         + rels_ref[2, k], 0.0)
        t = t + rk * wars_ref[0, k]
    aw = jnp.dot(h_blk, war_ref[R:R + D], preferred_element_type=f32)
    bw = jnp.dot(warhjt_ref[...], ht_ref[...], preferred_element_type=f32)
    tt = t + aw + bw + wars_ref[1, 0]

    m2 = (nei_ref[...] > 0) & (tt != 0.0)
    mx = jnp.max(jnp.where(m2, tt, NEG), axis=1, keepdims=True)
    w = jnp.where(m2, jnp.exp(tt - mx), 0.0)
    s = jnp.sum(w, axis=1, keepdims=True)
    p = (w / jnp.where(s > 0.0, s, 1.0)).astype(bf16)

    h_sum = jax.lax.dot_general(
        p, q, (((1,), (1,)), ((0,), (0,))),
        preferred_element_type=f32)

    c_out = jnp.dot(h_sum, wnei_ref[...], preferred_element_type=f32)
    c_out = c_out + wneib_ref[...] + c_ref[...]
    cout_ref[...] = c_out
    hout_ref[...] = og_ref[...] * jnp.tanh(c_out)


def kernel(corr_index, nei_index, nei_num, outgate, self_h, self_c,
           rel_W, rel_b, ngate_W, ngate_b, war_W, war_b, wnei_W, wnei_b):
    n = corr_index.shape[0]
    d = self_h.shape[1]
    ri = corr_index.shape[2]
    r = rel_W.shape[1]
    assert (n, d, ri, r) == (N, D, RI, R)
    f32 = jnp.float32

    ct = corr_index.transpose(2, 0, 1)

    rels = jnp.stack([rel_W[0], rel_W[1], rel_b])
    wars = jnp.zeros((2, r), f32).at[0].set(war_W[:r, 0]).at[1, 0].set(war_b[0])

    grid = (n // BI,)
    full = lambda shape: pl.BlockSpec(shape, lambda i: (0,) * len(shape))
    row_blk = lambda shape: pl.BlockSpec(shape, lambda i: (i,) + (0,) * (len(shape) - 1))
    ct_blk = pl.BlockSpec((RI, BI, n), lambda i: (0, i, 0))
    smem = lambda shape: pl.BlockSpec(shape, lambda i: (0,) * len(shape),
                                      memory_space=pltpu.SMEM)

    h_out, c_out = pl.pallas_call(
        _gcn_block,
        grid=grid,
        in_specs=[
            row_blk((BI, n, ri)),
            ct_blk,
            row_blk((BI, n)),
            full((n, d)),
            full((d, n)),
            row_blk((BI, d)),
            row_blk((BI, d)),
            full((ri, r)),
            full((1, r)),
            full((r + 2 * d, d)),
            full((1, d)),
            full((r + 2 * d, 1)),
            full((1, d)),
            smem((3, r)),
            smem((2, r)),
            full((d, d)),
            full((1, d)),
        ],
        out_specs=[row_blk((BI, d)), row_blk((BI, d))],
        out_shape=[
            jax.ShapeDtypeStruct((n, d), f32),
            jax.ShapeDtypeStruct((n, d), f32),
        ],
        compiler_params=pltpu.CompilerParams(
            dimension_semantics=("arbitrary",),
        ),
    )(corr_index, ct, nei_index, self_h, self_h.T, self_c, outgate,
      rel_W, rel_b.reshape(1, r), ngate_W, ngate_b.reshape(1, d),
      war_W, war_W[r + d:].T, rels, wars, wnei_W, wnei_b.reshape(1, d))

    return (outgate, h_out, c_out)
